# baseline (speedup 1.0000x reference)
import functools

import jax
import jax.numpy as jnp
from jax import lax
from jax.experimental import pallas as pl
from jax.experimental.pallas import tpu as pltpu
from jax.experimental.pallas import tpu_sc as plsc

NC = 2
NS = 16
L = 16
NW = NC * NS

D = 64
CHUNK = 64
NBUF = 8
G_LEAD = 4
O_LEAD = 6


def kernel(t, emb):
    A, B, Dp = t.shape
    assert Dp == D
    V, De = emb.shape
    assert De == D
    R = A * B
    assert R % NW == 0
    rows_per_w = R // NW
    assert rows_per_w % (CHUNK * NBUF) == 0
    n_chunks = rows_per_w // CHUNK
    n_groups = n_chunks // NBUF

    t2 = t.reshape(R, D)
    mesh = plsc.VectorSubcoreMesh(core_axis_name="c", subcore_axis_name="s")

    @functools.partial(
        pl.kernel,
        mesh=mesh,
        compiler_params=pltpu.CompilerParams(
            use_tc_tiling_on_sc=False, needs_layout_passes=False
        ),
        out_type=jax.ShapeDtypeStruct((R, 2 * D), jnp.float32),
        scratch_types=[
            pltpu.VMEM((NBUF, CHUNK, 2 * D), jnp.float32),
            pltpu.VMEM((NBUF, CHUNK, D), jnp.float32),
            pltpu.VMEM((NBUF, CHUNK), jnp.int32),
        ]
        + [pltpu.SemaphoreType.DMA] * (3 * NBUF),
    )
    def body(t_hbm, emb_hbm, out_hbm, stage, ebuf, iv, *sems):
        isem = sems[0:NBUF]
        gsem = sems[NBUF:2 * NBUF]
        osem = sems[2 * NBUF:3 * NBUF]

        wid = lax.axis_index("s") * NC + lax.axis_index("c")
        wbase = wid * rows_per_w

        lane = lax.iota(jnp.int32, L)
        col2 = jnp.full((L,), 2, jnp.int32)
        vmax = jnp.full((L,), V - 1, jnp.int32)
        zero = jnp.zeros((L,), jnp.int32)

        def in_copy(ci, b):
            return pltpu.make_async_copy(
                t_hbm.at[pl.ds(wbase + ci * CHUNK, CHUNK)],
                stage.at[b, :, pl.ds(0, D)],
                isem[b],
            )

        def gather_copy(b):
            return pltpu.make_async_copy(
                emb_hbm.at[iv.at[b]], ebuf.at[b], gsem[b]
            )

        def out_copy(ci, b):
            return pltpu.make_async_copy(
                stage.at[b], out_hbm.at[pl.ds(wbase + ci * CHUNK, CHUNK)],
                osem[b],
            )

        def emb_to_stage(sb):
            def copy_body(r, c2):
                for q in range(D // L):
                    stage[sb, r, pl.ds(D + q * L, L)] = (
                        ebuf[sb, r, pl.ds(q * L, L)]
                    )
                return c2

            lax.fori_loop(0, CHUNK, copy_body, 0)

        in_copy(0, 0).start()
        in_copy(1, 1).start()

        def group_body(g, carry):
            for b in range(NBUF):
                ci = g * NBUF + b

                in_copy(ci, b).wait()

                for j in range(CHUNK // L):
                    rows = lane + j * L
                    vals = plsc.load_gather(stage.at[b], [rows, col2])
                    idx = jnp.minimum(
                        jnp.maximum(vals.astype(jnp.int32), zero), vmax
                    )
                    iv[b, pl.ds(j * L, L)] = idx

                gather_copy(b).start()

                sb = (b - G_LEAD) % NBUF

                @pl.when(ci >= G_LEAD)
                def _():
                    gather_copy(sb).wait()
                    emb_to_stage(sb)
                    out_copy(ci - G_LEAD, sb).start()

                rb = (b + 2) % NBUF

                @pl.when(ci >= O_LEAD)
                def _():
                    out_copy(ci - O_LEAD, rb).wait()

                @pl.when(ci + 2 < n_chunks)
                def _():
                    in_copy(ci + 2, rb).start()
            return carry

        lax.fori_loop(0, n_groups, group_body, 0)

        for k in range(G_LEAD):
            c = n_chunks - G_LEAD + k
            bb = c % NBUF
            gather_copy(bb).wait()
            emb_to_stage(bb)
            out_copy(c, bb).start()
        for k in range(O_LEAD):
            c = n_chunks - O_LEAD + k
            out_copy(c, c % NBUF).wait()

    out2 = body(t2, emb)
    return out2.reshape(A, B, 2 * D)


# speedup vs baseline: 11.1433x; 11.1433x over previous
import functools

import jax
import jax.numpy as jnp
from jax import lax
from jax.experimental import pallas as pl
from jax.experimental.pallas import tpu as pltpu
from jax.experimental.pallas import tpu_sc as plsc

NC = 2
NS = 16
L = 16
NW = NC * NS

D = 64
CHUNK = 64
NBUF = 4


def kernel(t, emb):
    A, B, Dp = t.shape
    assert Dp == D
    V, De = emb.shape
    assert De == D
    R = A * B
    assert R % NW == 0
    rows_per_w = R // NW
    assert rows_per_w % (CHUNK * NBUF) == 0
    n_chunks = rows_per_w // CHUNK
    n_groups = n_chunks // NBUF

    t2 = t.reshape(R, D)
    mesh = plsc.VectorSubcoreMesh(core_axis_name="c", subcore_axis_name="s")

    @functools.partial(
        pl.kernel,
        mesh=mesh,
        compiler_params=pltpu.CompilerParams(
            use_tc_tiling_on_sc=False, needs_layout_passes=False
        ),
        out_type=jax.ShapeDtypeStruct((R, 2 * D), jnp.float32),
        scratch_types=[
            pltpu.VMEM((V, D), jnp.float32),
            pltpu.VMEM((NBUF, CHUNK, D), jnp.float32),
            pltpu.VMEM((NBUF, CHUNK, 2 * D), jnp.float32),
            pltpu.SemaphoreType.DMA,
        ]
        + [pltpu.SemaphoreType.DMA] * (2 * NBUF),
    )
    def body(t_hbm, emb_hbm, out_hbm, embl, tbuf, stage, esem, *sems):
        isem = sems[0:NBUF]
        osem = sems[NBUF:2 * NBUF]

        wid = lax.axis_index("s") * NC + lax.axis_index("c")
        wbase = wid * rows_per_w

        def in_copy(ci, b):
            return pltpu.make_async_copy(
                t_hbm.at[pl.ds(wbase + ci * CHUNK, CHUNK)], tbuf.at[b],
                isem[b],
            )

        def out_copy(ci, b):
            return pltpu.make_async_copy(
                stage.at[b], out_hbm.at[pl.ds(wbase + ci * CHUNK, CHUNK)],
                osem[b],
            )

        emb_load = pltpu.make_async_copy(emb_hbm, embl, esem)
        emb_load.start()
        in_copy(0, 0).start()
        in_copy(1, 1).start()
        emb_load.wait()

        def assemble(b):
            def row_body(r, c2):
                v = tbuf[b, r, pl.ds(0, L)][2]
                idx = jnp.minimum(
                    jnp.maximum(v.astype(jnp.int32), 0), V - 1
                )
                for q in range(D // L):
                    stage[b, r, pl.ds(q * L, L)] = tbuf[b, r, pl.ds(q * L, L)]
                    stage[b, r, pl.ds(D + q * L, L)] = embl[idx, pl.ds(q * L, L)]
                return c2

            lax.fori_loop(0, CHUNK, row_body, 0)

        def group_body(g, carry):
            for b in range(NBUF):
                ci = g * NBUF + b

                in_copy(ci, b).wait()

                @pl.when(ci >= NBUF)
                def _():
                    out_copy(ci - NBUF, b).wait()

                assemble(b)
                out_copy(ci, b).start()

                @pl.when(ci + 2 < n_chunks)
                def _():
                    in_copy(ci + 2, (b + 2) % NBUF).start()
            return carry

        lax.fori_loop(0, n_groups, group_body, 0)

        for k in range(NBUF):
            c = n_chunks - NBUF + k
            out_copy(c, c % NBUF).wait()

    out2 = body(t2, emb)
    return out2.reshape(A, B, 2 * D)


# speedup vs baseline: 11.7853x; 1.0576x over previous
import functools

import jax
import jax.numpy as jnp
from jax import lax
from jax.experimental import pallas as pl
from jax.experimental.pallas import tpu as pltpu
from jax.experimental.pallas import tpu_sc as plsc

NC = 2
NS = 16
L = 16
NW = NC * NS

D = 64
CHUNK = 64
NBUF = 4


def kernel(t, emb):
    A, B, Dp = t.shape
    assert Dp == D
    V, De = emb.shape
    assert De == D
    R = A * B
    assert R % NW == 0
    rows_per_w = R // NW
    assert rows_per_w % (CHUNK * NBUF) == 0
    n_chunks = rows_per_w // CHUNK
    n_groups = n_chunks // NBUF

    t2 = t.reshape(R, D)
    mesh = plsc.VectorSubcoreMesh(core_axis_name="c", subcore_axis_name="s")

    @functools.partial(
        pl.kernel,
        mesh=mesh,
        compiler_params=pltpu.CompilerParams(
            use_tc_tiling_on_sc=False, needs_layout_passes=False
        ),
        out_type=jax.ShapeDtypeStruct((R, 2 * D), jnp.float32),
        scratch_types=[
            pltpu.VMEM((V, D), jnp.float32),
            pltpu.VMEM((NBUF, CHUNK, D), jnp.float32),
            pltpu.VMEM((NBUF, CHUNK, 2 * D), jnp.float32),
            pltpu.SemaphoreType.DMA,
        ]
        + [pltpu.SemaphoreType.DMA] * (2 * NBUF),
    )
    def body(t_hbm, emb_hbm, out_hbm, embl, tbuf, stage, esem, *sems):
        isem = sems[0:NBUF]
        osem = sems[NBUF:2 * NBUF]

        wid = lax.axis_index("s") * NC + lax.axis_index("c")
        wbase = wid * rows_per_w

        def in_copy(ci, b):
            return pltpu.make_async_copy(
                t_hbm.at[pl.ds(wbase + ci * CHUNK, CHUNK)], tbuf.at[b],
                isem[b],
            )

        def out_copy(ci, b):
            return pltpu.make_async_copy(
                stage.at[b], out_hbm.at[pl.ds(wbase + ci * CHUNK, CHUNK)],
                osem[b],
            )

        emb_load = pltpu.make_async_copy(emb_hbm, embl, esem)
        emb_load.start()
        in_copy(0, 0).start()
        in_copy(1, 1).start()
        emb_load.wait()

        lane = lax.iota(jnp.int32, L)
        col2 = jnp.full((L,), 2, jnp.int32)

        def assemble(b):
            def row_body(r, c2):
                rvec = jnp.full((L,), r, jnp.int32)
                vals = plsc.load_gather(tbuf.at[b], [rvec, col2])
                idxv = jnp.minimum(
                    jnp.maximum(vals.astype(jnp.int32), 0), V - 1
                )
                for q in range(D // L):
                    stage[b, r, pl.ds(q * L, L)] = tbuf[b, r, pl.ds(q * L, L)]
                    stage[b, r, pl.ds(D + q * L, L)] = plsc.load_gather(
                        embl, [idxv, lane + q * L]
                    )
                return c2

            lax.fori_loop(0, CHUNK, row_body, 0)

        def group_body(g, carry):
            for b in range(NBUF):
                ci = g * NBUF + b

                in_copy(ci, b).wait()

                @pl.when(ci >= NBUF)
                def _():
                    out_copy(ci - NBUF, b).wait()

                assemble(b)
                out_copy(ci, b).start()

                @pl.when(ci + 2 < n_chunks)
                def _():
                    in_copy(ci + 2, (b + 2) % NBUF).start()
            return carry

        lax.fori_loop(0, n_groups, group_body, 0)

        for k in range(NBUF):
            c = n_chunks - NBUF + k
            out_copy(c, c % NBUF).wait()

    out2 = body(t2, emb)
    return out2.reshape(A, B, 2 * D)
